# baseline (speedup 1.0000x reference)
import functools

import jax
import jax.numpy as jnp
from jax import lax
from jax.experimental import pallas as pl
from jax.experimental.pallas import tpu as pltpu
from jax.experimental.pallas import tpu_sc as plsc

N = 10000
E = 320000
D = 128
H = 128
C = 40
L = 3
G = 64

NW = 32
NCHK = N // 8
NVIRT = 10240
ACCR = 322
CAP = 12288
EB = 128

_BOUNDS = []
for _c in range(2):
    _base = _c * (E // 2)
    if _c == 1:
        _BOUNDS.append(_base)
    for _k in range(1, 16):
        _BOUNDS.append(_base + 240 * (42 * min(_k, 11) + 41 * max(0, _k - 11)))


def _edge_metadata(src, dst):
    ds = jnp.sort(dst)
    rows16 = lax.broadcasted_iota(jnp.int32, (NW, 16), 0)
    cols16 = lax.broadcasted_iota(jnp.int32, (NW, 16), 1)
    wiota = lax.broadcasted_iota(jnp.int32, (NW,), 0)
    dst2 = dst
    blid = jnp.full((NW, 16), -1, jnp.int32)
    nslot = jnp.zeros((NW,), jnp.int32)
    for b in _BOUNDS:
        n_b = ds[b]
        s0 = jnp.searchsorted(ds, n_b, side="left").astype(jnp.int32)
        e0 = jnp.searchsorted(ds, n_b, side="right").astype(jnp.int32)
        c_b = jnp.int32(b) - s0
        valid = (c_b > 0) & (jnp.int32(b) < e0)
        w_b = (n_b >> 3) & 31
        s_b = nslot[w_b]
        vid = NVIRT + 2 * w_b + s_b
        rank = jnp.cumsum((dst == n_b).astype(jnp.int32))
        repl = valid & (dst == n_b) & (rank > c_b)
        dst2 = jnp.where(repl, vid.astype(jnp.int32), dst2)
        lid = ((n_b >> 8) << 3) | (n_b & 7)
        hit = valid & (rows16 == w_b) & (cols16 == s_b)
        blid = jnp.where(hit, lid, blid)
        nslot = nslot + jnp.where(valid & (wiota == w_b), 1, 0)

    is_v = dst2 >= NVIRT
    owner = jnp.where(is_v, (dst2 - NVIRT) >> 1, (dst2 >> 3) & 31)
    lid_all = jnp.where(is_v, 320 + (dst2 & 1), ((dst2 >> 8) << 3) | (dst2 & 7))
    perm = jnp.argsort(owner, stable=True)
    starts_w = jnp.searchsorted(owner[perm], jnp.arange(NW + 1)).astype(jnp.int32)
    cnts = starts_w[1:] - starts_w[:-1]
    cntp = jnp.concatenate([cnts, jnp.zeros((16,), jnp.int32)])
    j = jnp.arange(NW * CAP, dtype=jnp.int32)
    ow = j // CAP
    pos = j % CAP
    idx = jnp.clip(starts_w[ow] + pos, 0, E - 1)
    validp = pos < cnts[ow]
    srcp = jnp.where(validp, src[perm][idx], 0)
    lidp = jnp.where(validp, lid_all[perm][idx], 0)
    return srcp, lidp, cntp, blid


def _sc_agg_body(x_hbm, srcp_hbm, lidp_hbm, cnt_hbm, blid_hbm, out_hbm,
                 sidx, lidx, rows, acc, blv, cntv, sem):
    cid = lax.axis_index("c")
    sid = lax.axis_index("s")
    w = sid * 2 + cid

    zero16 = jnp.zeros((16,), jnp.float32)

    def _zacc(i, carry):
        acc[i // 8, pl.ds((i % 8) * 16, 16)] = zero16
        return carry
    lax.fori_loop(0, ACCR * 8, _zacc, 0)

    pltpu.sync_copy(blid_hbm, blv)
    pltpu.sync_copy(cnt_hbm, cntv)
    cnt = cntv[pl.ds(w, 16)][0]

    nb = (cnt + EB - 1) // EB

    def _batch(b, carry):
        off = w * CAP + b * EB
        pltpu.sync_copy(srcp_hbm.at[pl.ds(off, EB)], sidx)
        pltpu.sync_copy(lidp_hbm.at[pl.ds(off, EB)], lidx.at[pl.ds(0, EB)])
        pltpu.async_copy(x_hbm.at[sidx], rows, sem).wait()
        n_add = jnp.minimum(cnt - b * EB, EB)

        def _add(k, c2):
            lid = lidx[pl.ds(k, 16)][0]
            for f in range(8):
                acc[lid, pl.ds(f * 16, 16)] = (
                    acc[lid, pl.ds(f * 16, 16)]
                    + rows[k, pl.ds(f * 16, 16)])
            return c2
        lax.fori_loop(0, n_add, _add, 0)
        return carry
    lax.fori_loop(0, nb, _batch, 0)

    blrow = blv[w, pl.ds(0, 16)]
    for s2 in range(2):
        bl = blrow[s2]

        @pl.when(bl >= 0)
        def _():
            for f in range(8):
                acc[bl, pl.ds(f * 16, 16)] = (
                    acc[bl, pl.ds(f * 16, 16)]
                    + acc[320 + s2, pl.ds(f * 16, 16)])

    for jj in range(40):
        c = jj * NW + w

        @pl.when(c < NCHK)
        def _():
            pltpu.sync_copy(acc.at[pl.ds(jj * 8, 8)],
                            out_hbm.at[pl.ds(c * 8, 8)])


@functools.lru_cache(maxsize=1)
def _get_sc_agg():
    return pl.kernel(
        _sc_agg_body,
        out_type=jax.ShapeDtypeStruct((N, D), jnp.float32),
        mesh=plsc.VectorSubcoreMesh(core_axis_name="c", subcore_axis_name="s"),
        scratch_types=[
            pltpu.VMEM((EB,), jnp.int32),
            pltpu.VMEM((EB + 16,), jnp.int32),
            pltpu.VMEM((EB, D), jnp.float32),
            pltpu.VMEM((ACCR, D), jnp.float32),
            pltpu.VMEM((NW, 16), jnp.int32),
            pltpu.VMEM((48,), jnp.int32),
            pltpu.SemaphoreType.DMA,
        ],
    )


def _mlp_block(h, w1, b1, g, be, w2, b2):
    h = jnp.dot(h, w1, preferred_element_type=jnp.float32) + b1
    m = jnp.mean(h, axis=0, keepdims=True)
    v = jnp.mean((h - m) ** 2, axis=0, keepdims=True)
    h = g * (h - m) / jnp.sqrt(v + 1e-5) + be
    h = jnp.maximum(h, 0.0)
    return jnp.dot(h, w2, preferred_element_type=jnp.float32) + b2


def _layer_tc_body(agg_ref, x_ref,
                   wr1, br1, gr1, ber1, wr2, br2,
                   wx1, bx1, gx1, bex1, wx2, bx2, out_ref):
    y = _mlp_block(agg_ref[...], wr1[...], br1[...], gr1[...], ber1[...],
                   wr2[...], br2[...])
    y = y + _mlp_block(x_ref[...], wx1[...], bx1[...], gx1[...], bex1[...],
                       wx2[...], bx2[...])
    out_ref[...] = jnp.maximum(y, 0.0)


def _layer_tc(agg, x, *params):
    return pl.pallas_call(
        _layer_tc_body,
        out_shape=jax.ShapeDtypeStruct((N, D), jnp.float32),
    )(agg, x, *params)


def _pool_mlp_body(x_ref, batch_ref, w1, b1, g, be, w2, b2, out_ref):
    b = batch_ref[...]
    gids = lax.broadcasted_iota(jnp.int32, (1, G), 1)
    oh = (b == gids).astype(jnp.float32)
    pooled = lax.dot_general(oh, x_ref[...], (((0,), (0,)), ((), ())),
                             preferred_element_type=jnp.float32,
                             precision=lax.Precision.HIGHEST)
    out_ref[...] = _mlp_block(pooled, w1[...], b1[...], g[...], be[...],
                              w2[...], b2[...])


def _pool_mlp(x, batch2, *params):
    return pl.pallas_call(
        _pool_mlp_body,
        out_shape=jax.ShapeDtypeStruct((G, C), jnp.float32),
    )(x, batch2, *params)


def kernel(x, edge_index, batch, rel_W1, rel_b1, rel_g1, rel_be1, rel_W2, rel_b2,
           root_W1, root_b1, root_g1, root_be1, root_W2, root_b2,
           mlp_W1, mlp_b1, mlp_g, mlp_be, mlp_W2, mlp_b2):
    src = edge_index[0]
    dst = edge_index[1]
    srcp, lidp, cntp, blid = _edge_metadata(src, dst)
    batch2 = batch.reshape(N, 1)
    r2 = lambda a: a.reshape(1, -1)
    h = x
    for l in range(L):
        agg = _get_sc_agg()(h, srcp, lidp, cntp, blid)
        h = _layer_tc(
            agg, h,
            rel_W1[l], r2(rel_b1[l]), r2(rel_g1[l]), r2(rel_be1[l]),
            rel_W2[l], r2(rel_b2[l]),
            root_W1[l], r2(root_b1[l]), r2(root_g1[l]), r2(root_be1[l]),
            root_W2[l], r2(root_b2[l]),
        )
    return _pool_mlp(h, batch2, mlp_W1, r2(mlp_b1), r2(mlp_g), r2(mlp_be),
                     mlp_W2, r2(mlp_b2))
